# Initial kernel scaffold; baseline (speedup 1.0000x reference)
import jax
import jax.numpy as jnp
from jax.experimental import pallas as pl


def kernel(features, edge_index, W0, b0, W1, b1, W2, b2, noise):
    raise NotImplementedError("write your pallas kernel here")


# speedup vs baseline: 1.1831x; 1.1831x over previous
import jax
import jax.numpy as jnp
from jax.experimental import pallas as pl
from jax.experimental.pallas import tpu as pltpu

N_NODES = 10000


def _decoder_body(zi_ref, zj_ref, out_ref):
    acc = jax.lax.dot_general(
        zi_ref[...], zj_ref[...], (((1,), (1,)), ((), ())),
        preferred_element_type=jnp.float32)
    out_ref[...] = jax.nn.sigmoid(acc)


def _decoder(z):
    n = z.shape[0]
    BI = BJ = 1024
    gi = pl.cdiv(n, BI)
    return pl.pallas_call(
        _decoder_body,
        grid=(gi, gi),
        in_specs=[
            pl.BlockSpec((BI, z.shape[1]), lambda i, j: (i, 0)),
            pl.BlockSpec((BJ, z.shape[1]), lambda i, j: (j, 0)),
        ],
        out_specs=pl.BlockSpec((BI, BJ), lambda i, j: (i, j)),
        out_shape=jax.ShapeDtypeStruct((n, n), jnp.float32),
    )(z, z)


def _segment_sum(vals, dst, n):
    return jnp.zeros((n, vals.shape[1]), dtype=vals.dtype).at[dst].add(vals)


def kernel(features, edge_index, W0, b0, W1, b1, W2, b2, noise):
    n = features.shape[0]
    src = edge_index[0].astype(jnp.int32)
    dst = edge_index[1].astype(jnp.int32)

    ones = jnp.ones((src.shape[0],), dtype=jnp.float32)
    deg_out = jnp.zeros((n,), jnp.float32).at[src].add(ones)
    deg_in = jnp.zeros((n,), jnp.float32).at[dst].add(ones)
    ns = jax.lax.rsqrt(jnp.maximum(deg_out, 1.0))
    nd = jax.lax.rsqrt(jnp.maximum(deg_in, 1.0))

    m0 = (features @ W0) * ns[:, None]
    agg0 = _segment_sum(m0[src], dst, n)
    h = jax.nn.relu(agg0 * nd[:, None] + b0)

    h1n = h * ns[:, None]
    aggH = _segment_sum(h1n[src], dst, n)
    mean = (aggH @ W1) * nd[:, None] + b1
    log_std = (aggH @ W2) * nd[:, None] + b2
    z = mean + noise * jnp.exp(log_std)

    adj_rec = _decoder(z)
    return (adj_rec, z)


# speedup vs baseline: 6.7443x; 5.7006x over previous
import functools

import jax
import jax.numpy as jnp
from jax import lax
from jax.experimental import pallas as pl
from jax.experimental.pallas import tpu as pltpu
from jax.experimental.pallas import tpu_sc as plsc

N = 10000
N_PAD = 10240
E = 320000
NC, NS = 2, 16
NW = NC * NS
EPW = E // NW
CB = 80
CK = EPW // CB
STRIPE = N_PAD // NS

_sc_mesh = plsc.VectorSubcoreMesh(core_axis_name="c", subcore_axis_name="s")


def _deg_body(srcr, dstr, zdeg, out, idx_s, idx_d, ones_v, dout_sh, din_sh,
              sem):
    c = lax.axis_index("c")
    s = lax.axis_index("s")
    wid = s * NC + c
    for i in range(CB // 16):
        ones_v[pl.ds(i * 16, 16)] = jnp.ones((16,), jnp.float32)
    pltpu.sync_copy(zdeg.at[pl.ds(s * STRIPE, STRIPE)],
                    dout_sh.at[pl.ds(s * STRIPE, STRIPE)])
    pltpu.sync_copy(zdeg.at[pl.ds(s * STRIPE, STRIPE)],
                    din_sh.at[pl.ds(s * STRIPE, STRIPE)])
    pltpu.async_copy(srcr.at[wid], idx_s, sem).wait()
    pltpu.async_copy(dstr.at[wid], idx_d, sem).wait()
    plsc.subcore_barrier()

    def body(j, carry):
        pltpu.sync_copy(ones_v, dout_sh.at[idx_s.at[j]], add=True)
        pltpu.sync_copy(ones_v, din_sh.at[idx_d.at[j]], add=True)
        return carry

    lax.fori_loop(0, CK, body, 0)
    plsc.subcore_barrier()
    pltpu.sync_copy(dout_sh.at[pl.ds(s * STRIPE, STRIPE)],
                    out.at[c, 0, pl.ds(s * STRIPE, STRIPE)])
    pltpu.sync_copy(din_sh.at[pl.ds(s * STRIPE, STRIPE)],
                    out.at[c, 1, pl.ds(s * STRIPE, STRIPE)])


@functools.partial(
    pl.kernel,
    out_type=jax.ShapeDtypeStruct((NC, 2, N_PAD), jnp.float32),
    mesh=_sc_mesh,
    compiler_params=pltpu.CompilerParams(use_tc_tiling_on_sc=False),
    scratch_types=[
        pltpu.VMEM((CK, CB), jnp.int32),
        pltpu.VMEM((CK, CB), jnp.int32),
        pltpu.VMEM((CB,), jnp.float32),
        pltpu.VMEM_SHARED((N_PAD,), jnp.float32),
        pltpu.VMEM_SHARED((N_PAD,), jnp.float32),
        pltpu.SemaphoreType.DMA,
    ],
)
def _sc_degrees(srcr, dstr, zdeg, out, idx_s, idx_d, ones_v, dout_sh, din_sh,
                sem):
    _deg_body(srcr, dstr, zdeg, out, idx_s, idx_d, ones_v, dout_sh, din_sh,
              sem)


def _agg_body(table, srcr, dstr, zfeat, out, idx_s, idx_d, rows_v, agg_sh, sem):
    c = lax.axis_index("c")
    s = lax.axis_index("s")
    wid = s * NC + c
    pltpu.sync_copy(zfeat.at[pl.ds(s * STRIPE, STRIPE)],
                    agg_sh.at[pl.ds(s * STRIPE, STRIPE)])
    pltpu.async_copy(srcr.at[wid], idx_s, sem).wait()
    pltpu.async_copy(dstr.at[wid], idx_d, sem).wait()
    plsc.subcore_barrier()

    def body(j, carry):
        pltpu.async_copy(table.at[idx_s.at[j]], rows_v, sem).wait()
        pltpu.sync_copy(rows_v, agg_sh.at[idx_d.at[j]], add=True)
        return carry

    lax.fori_loop(0, CK, body, 0)
    plsc.subcore_barrier()
    pltpu.sync_copy(agg_sh.at[pl.ds(s * STRIPE, STRIPE)],
                    out.at[c, pl.ds(s * STRIPE, STRIPE)])


def _make_sc_agg(d):
    @functools.partial(
        pl.kernel,
        out_type=jax.ShapeDtypeStruct((NC, N_PAD, d), jnp.float32),
        mesh=_sc_mesh,
        compiler_params=pltpu.CompilerParams(use_tc_tiling_on_sc=False),
        scratch_types=[
            pltpu.VMEM((CK, CB), jnp.int32),
            pltpu.VMEM((CK, CB), jnp.int32),
            pltpu.VMEM((CB, d), jnp.float32),
            pltpu.VMEM_SHARED((N_PAD, d), jnp.float32),
            pltpu.SemaphoreType.DMA,
        ],
    )
    def _sc_agg(table, srcr, dstr, zfeat, out, idx_s, idx_d, rows_v, agg_sh, sem):
        _agg_body(table, srcr, dstr, zfeat, out, idx_s, idx_d, rows_v, agg_sh, sem)

    return _sc_agg


_sc_agg64 = _make_sc_agg(64)


def _norms(degp_ref):
    deg_out = degp_ref[0, 0] + degp_ref[1, 0]
    deg_in = degp_ref[0, 1] + degp_ref[1, 1]
    ns = lax.rsqrt(jnp.maximum(deg_out, 1.0))
    nd = lax.rsqrt(jnp.maximum(deg_in, 1.0))
    return ns, nd


def _tc1_body(x_ref, w_ref, degp_ref, y_ref):
    ns, _ = _norms(degp_ref)
    y_ref[...] = jnp.dot(x_ref[...], w_ref[...],
                         preferred_element_type=jnp.float32) * ns


def _tc1(features, W0, degp):
    B = 1000
    return pl.pallas_call(
        _tc1_body,
        grid=(N // B,),
        in_specs=[
            pl.BlockSpec((B, 128), lambda i: (i, 0)),
            pl.BlockSpec((128, 64), lambda i: (0, 0)),
            pl.BlockSpec((NC, 2, B, 1), lambda i: (0, 0, i, 0)),
        ],
        out_specs=pl.BlockSpec((B, 64), lambda i: (i, 0)),
        out_shape=jax.ShapeDtypeStruct((N, 64), jnp.float32),
    )(features, W0, degp)


def _tc2_body(agg_ref, b0_ref, degp_ref, y_ref):
    ns, nd = _norms(degp_ref)
    h = jax.nn.relu((agg_ref[0] + agg_ref[1]) * nd + b0_ref[...])
    y_ref[...] = h * ns


def _tc2(agg0, b0, degp):
    B = 1000
    return pl.pallas_call(
        _tc2_body,
        grid=(N // B,),
        in_specs=[
            pl.BlockSpec((NC, B, 64), lambda i: (0, i, 0)),
            pl.BlockSpec((1, 64), lambda i: (0, 0)),
            pl.BlockSpec((NC, 2, B, 1), lambda i: (0, 0, i, 0)),
        ],
        out_specs=pl.BlockSpec((B, 64), lambda i: (i, 0)),
        out_shape=jax.ShapeDtypeStruct((N, 64), jnp.float32),
    )(agg0, b0, degp)


def _tc3_body(agg_ref, w1_ref, w2_ref, b1_ref, b2_ref, degp_ref, noise_ref,
              z_ref):
    _, nd = _norms(degp_ref)
    a = agg_ref[0] + agg_ref[1]
    mean = jnp.dot(a, w1_ref[...], preferred_element_type=jnp.float32) * nd \
        + b1_ref[...]
    log_std = jnp.dot(a, w2_ref[...], preferred_element_type=jnp.float32) * nd \
        + b2_ref[...]
    z_ref[...] = mean + noise_ref[...] * jnp.exp(log_std)


def _tc3(aggH, W1, W2, b1, b2, degp, noise):
    B = 1000
    return pl.pallas_call(
        _tc3_body,
        grid=(N // B,),
        in_specs=[
            pl.BlockSpec((NC, B, 64), lambda i: (0, i, 0)),
            pl.BlockSpec((64, 32), lambda i: (0, 0)),
            pl.BlockSpec((64, 32), lambda i: (0, 0)),
            pl.BlockSpec((1, 32), lambda i: (0, 0)),
            pl.BlockSpec((1, 32), lambda i: (0, 0)),
            pl.BlockSpec((NC, 2, B, 1), lambda i: (0, 0, i, 0)),
            pl.BlockSpec((B, 32), lambda i: (i, 0)),
        ],
        out_specs=pl.BlockSpec((B, 32), lambda i: (i, 0)),
        out_shape=jax.ShapeDtypeStruct((N, 32), jnp.float32),
    )(aggH, W1, W2, b1, b2, degp, noise)


def _decoder_body(zi_ref, zj_ref, out_ref):
    acc = lax.dot_general(
        zi_ref[...], zj_ref[...], (((1,), (1,)), ((), ())),
        preferred_element_type=jnp.float32)
    out_ref[...] = jax.nn.sigmoid(acc)


def _decoder(z):
    BI = BJ = 1024
    gi = pl.cdiv(N, BI)
    return pl.pallas_call(
        _decoder_body,
        grid=(gi, gi),
        in_specs=[
            pl.BlockSpec((BI, 32), lambda i, j: (i, 0)),
            pl.BlockSpec((BJ, 32), lambda i, j: (j, 0)),
        ],
        out_specs=pl.BlockSpec((BI, BJ), lambda i, j: (i, j)),
        out_shape=jax.ShapeDtypeStruct((N, N), jnp.float32),
    )(z, z)


def kernel(features, edge_index, W0, b0, W1, b1, W2, b2, noise):
    srcr = edge_index[0].astype(jnp.int32).reshape(NW, CK, CB)
    dstr = edge_index[1].astype(jnp.int32).reshape(NW, CK, CB)
    zdeg = jnp.zeros((N_PAD,), jnp.float32)
    zfeat = jnp.zeros((N_PAD, 64), jnp.float32)

    degp = _sc_degrees(srcr, dstr, zdeg)
    degp = degp.reshape(NC, 2, N_PAD, 1)

    m0 = _tc1(features, W0, degp)
    agg0 = _sc_agg64(m0, srcr, dstr, zfeat)
    h1n = _tc2(agg0, b0.reshape(1, 64), degp)
    aggH = _sc_agg64(h1n, srcr, dstr, zfeat)
    z = _tc3(aggH, W1, W2, b1.reshape(1, 32), b2.reshape(1, 32), degp, noise)
    adj_rec = _decoder(z)
    return (adj_rec, z)


# speedup vs baseline: 8.0413x; 1.1923x over previous
import functools

import jax
import jax.numpy as jnp
from jax import lax
from jax.experimental import pallas as pl
from jax.experimental.pallas import tpu as pltpu
from jax.experimental.pallas import tpu_sc as plsc

N = 10000
N_PAD = 10240
E = 320000
NC, NS = 2, 16
NW = NC * NS
EPW = E // NW
CB = 80
CK = EPW // CB
STRIPE = N_PAD // NS

_sc_mesh = plsc.VectorSubcoreMesh(core_axis_name="c", subcore_axis_name="s")


def _deg_body(srcr, dstr, zdeg, out, idx_s, idx_d, ones_v, dout_sh, din_sh,
              sem):
    c = lax.axis_index("c")
    s = lax.axis_index("s")
    wid = s * NC + c
    for i in range(CB // 16):
        ones_v[pl.ds(i * 16, 16)] = jnp.ones((16,), jnp.float32)
    pltpu.sync_copy(zdeg.at[pl.ds(s * STRIPE, STRIPE)],
                    dout_sh.at[pl.ds(s * STRIPE, STRIPE)])
    pltpu.sync_copy(zdeg.at[pl.ds(s * STRIPE, STRIPE)],
                    din_sh.at[pl.ds(s * STRIPE, STRIPE)])
    pltpu.async_copy(srcr.at[wid], idx_s, sem).wait()
    pltpu.async_copy(dstr.at[wid], idx_d, sem).wait()
    plsc.subcore_barrier()

    def body(j, carry):
        pltpu.sync_copy(ones_v, dout_sh.at[idx_s.at[j]], add=True)
        pltpu.sync_copy(ones_v, din_sh.at[idx_d.at[j]], add=True)
        return carry

    lax.fori_loop(0, CK, body, 0)
    plsc.subcore_barrier()
    pltpu.sync_copy(dout_sh.at[pl.ds(s * STRIPE, STRIPE)],
                    out.at[c, 0, pl.ds(s * STRIPE, STRIPE)])
    pltpu.sync_copy(din_sh.at[pl.ds(s * STRIPE, STRIPE)],
                    out.at[c, 1, pl.ds(s * STRIPE, STRIPE)])


@functools.partial(
    pl.kernel,
    out_type=jax.ShapeDtypeStruct((NC, 2, N_PAD), jnp.float32),
    mesh=_sc_mesh,
    compiler_params=pltpu.CompilerParams(use_tc_tiling_on_sc=False),
    scratch_types=[
        pltpu.VMEM((CK, CB), jnp.int32),
        pltpu.VMEM((CK, CB), jnp.int32),
        pltpu.VMEM((CB,), jnp.float32),
        pltpu.VMEM_SHARED((N_PAD,), jnp.float32),
        pltpu.VMEM_SHARED((N_PAD,), jnp.float32),
        pltpu.SemaphoreType.DMA,
    ],
)
def _sc_degrees(srcr, dstr, zdeg, out, idx_s, idx_d, ones_v, dout_sh, din_sh,
                sem):
    _deg_body(srcr, dstr, zdeg, out, idx_s, idx_d, ones_v, dout_sh, din_sh,
              sem)


def _agg_body(table, srcr, dstr, zfeat, out, idx_s, idx_d, rows_a, rows_b,
              agg_sh, sem_a, sem_b):
    c = lax.axis_index("c")
    s = lax.axis_index("s")
    wid = s * NC + c
    pltpu.sync_copy(zfeat.at[pl.ds(s * STRIPE, STRIPE)],
                    agg_sh.at[pl.ds(s * STRIPE, STRIPE)])
    pltpu.async_copy(srcr.at[wid], idx_s, sem_a).wait()
    pltpu.async_copy(dstr.at[wid], idx_d, sem_a).wait()
    plsc.subcore_barrier()

    pltpu.async_copy(table.at[idx_s.at[0]], rows_a, sem_a)

    def body(j, carry):
        @pl.when(j % 2 == 0)
        def _():
            pltpu.make_async_copy(table.at[idx_s.at[j]], rows_a, sem_a).wait()

            @pl.when(j + 1 < CK)
            def _():
                pltpu.async_copy(table.at[idx_s.at[j + 1]], rows_b, sem_b)

            pltpu.sync_copy(rows_a, agg_sh.at[idx_d.at[j]], add=True)

        @pl.when(j % 2 == 1)
        def _():
            pltpu.make_async_copy(table.at[idx_s.at[j]], rows_b, sem_b).wait()

            @pl.when(j + 1 < CK)
            def _():
                pltpu.async_copy(table.at[idx_s.at[j + 1]], rows_a, sem_a)

            pltpu.sync_copy(rows_b, agg_sh.at[idx_d.at[j]], add=True)

        return carry

    lax.fori_loop(0, CK, body, 0)
    plsc.subcore_barrier()
    pltpu.sync_copy(agg_sh.at[pl.ds(s * STRIPE, STRIPE)],
                    out.at[c, pl.ds(s * STRIPE, STRIPE)])


def _make_sc_agg(d):
    @functools.partial(
        pl.kernel,
        out_type=jax.ShapeDtypeStruct((NC, N_PAD, d), jnp.float32),
        mesh=_sc_mesh,
        compiler_params=pltpu.CompilerParams(use_tc_tiling_on_sc=False),
        scratch_types=[
            pltpu.VMEM((CK, CB), jnp.int32),
            pltpu.VMEM((CK, CB), jnp.int32),
            pltpu.VMEM((CB, d), jnp.float32),
            pltpu.VMEM((CB, d), jnp.float32),
            pltpu.VMEM_SHARED((N_PAD, d), jnp.float32),
            pltpu.SemaphoreType.DMA,
            pltpu.SemaphoreType.DMA,
        ],
    )
    def _sc_agg(table, srcr, dstr, zfeat, out, idx_s, idx_d, rows_a, rows_b,
                agg_sh, sem_a, sem_b):
        _agg_body(table, srcr, dstr, zfeat, out, idx_s, idx_d, rows_a, rows_b,
                  agg_sh, sem_a, sem_b)

    return _sc_agg


_sc_agg64 = _make_sc_agg(64)


def _norms(degp_ref):
    deg_out = degp_ref[0, 0] + degp_ref[1, 0]
    deg_in = degp_ref[0, 1] + degp_ref[1, 1]
    ns = lax.rsqrt(jnp.maximum(deg_out, 1.0))
    nd = lax.rsqrt(jnp.maximum(deg_in, 1.0))
    return ns, nd


def _tc1_body(x_ref, w_ref, degp_ref, y_ref):
    ns, _ = _norms(degp_ref)
    y_ref[...] = jnp.dot(x_ref[...], w_ref[...],
                         preferred_element_type=jnp.float32) * ns


def _tc1(features, W0, degp):
    B = 1000
    return pl.pallas_call(
        _tc1_body,
        grid=(N // B,),
        in_specs=[
            pl.BlockSpec((B, 128), lambda i: (i, 0)),
            pl.BlockSpec((128, 64), lambda i: (0, 0)),
            pl.BlockSpec((NC, 2, B, 1), lambda i: (0, 0, i, 0)),
        ],
        out_specs=pl.BlockSpec((B, 64), lambda i: (i, 0)),
        out_shape=jax.ShapeDtypeStruct((N, 64), jnp.float32),
    )(features, W0, degp)


def _tc2_body(agg_ref, b0_ref, degp_ref, y_ref):
    ns, nd = _norms(degp_ref)
    h = jax.nn.relu((agg_ref[0] + agg_ref[1]) * nd + b0_ref[...])
    y_ref[...] = h * ns


def _tc2(agg0, b0, degp):
    B = 1000
    return pl.pallas_call(
        _tc2_body,
        grid=(N // B,),
        in_specs=[
            pl.BlockSpec((NC, B, 64), lambda i: (0, i, 0)),
            pl.BlockSpec((1, 64), lambda i: (0, 0)),
            pl.BlockSpec((NC, 2, B, 1), lambda i: (0, 0, i, 0)),
        ],
        out_specs=pl.BlockSpec((B, 64), lambda i: (i, 0)),
        out_shape=jax.ShapeDtypeStruct((N, 64), jnp.float32),
    )(agg0, b0, degp)


def _tc3_body(agg_ref, w1_ref, w2_ref, b1_ref, b2_ref, degp_ref, noise_ref,
              z_ref):
    _, nd = _norms(degp_ref)
    a = agg_ref[0] + agg_ref[1]
    mean = jnp.dot(a, w1_ref[...], preferred_element_type=jnp.float32) * nd \
        + b1_ref[...]
    log_std = jnp.dot(a, w2_ref[...], preferred_element_type=jnp.float32) * nd \
        + b2_ref[...]
    z_ref[...] = mean + noise_ref[...] * jnp.exp(log_std)


def _tc3(aggH, W1, W2, b1, b2, degp, noise):
    B = 1000
    return pl.pallas_call(
        _tc3_body,
        grid=(N // B,),
        in_specs=[
            pl.BlockSpec((NC, B, 64), lambda i: (0, i, 0)),
            pl.BlockSpec((64, 32), lambda i: (0, 0)),
            pl.BlockSpec((64, 32), lambda i: (0, 0)),
            pl.BlockSpec((1, 32), lambda i: (0, 0)),
            pl.BlockSpec((1, 32), lambda i: (0, 0)),
            pl.BlockSpec((NC, 2, B, 1), lambda i: (0, 0, i, 0)),
            pl.BlockSpec((B, 32), lambda i: (i, 0)),
        ],
        out_specs=pl.BlockSpec((B, 32), lambda i: (i, 0)),
        out_shape=jax.ShapeDtypeStruct((N, 32), jnp.float32),
    )(aggH, W1, W2, b1, b2, degp, noise)


def _decoder_body(zi_ref, zj_ref, out_ref):
    acc = lax.dot_general(
        zi_ref[...], zj_ref[...], (((1,), (1,)), ((), ())),
        preferred_element_type=jnp.float32)
    out_ref[...] = jax.nn.sigmoid(acc)


def _decoder(z):
    BI = 400
    gi = N // BI
    return pl.pallas_call(
        _decoder_body,
        grid=(gi,),
        in_specs=[
            pl.BlockSpec((BI, 32), lambda i: (i, 0)),
            pl.BlockSpec((N, 32), lambda i: (0, 0)),
        ],
        out_specs=pl.BlockSpec((BI, N), lambda i: (i, 0)),
        out_shape=jax.ShapeDtypeStruct((N, N), jnp.float32),
    )(z, z)


def kernel(features, edge_index, W0, b0, W1, b1, W2, b2, noise):
    srcr = edge_index[0].astype(jnp.int32).reshape(NW, CK, CB)
    dstr = edge_index[1].astype(jnp.int32).reshape(NW, CK, CB)
    zdeg = jnp.zeros((N_PAD,), jnp.float32)
    zfeat = jnp.zeros((N_PAD, 64), jnp.float32)

    degp = _sc_degrees(srcr, dstr, zdeg)
    degp = degp.reshape(NC, 2, N_PAD, 1)

    m0 = _tc1(features, W0, degp)
    agg0 = _sc_agg64(m0, srcr, dstr, zfeat)
    h1n = _tc2(agg0, b0.reshape(1, 64), degp)
    aggH = _sc_agg64(h1n, srcr, dstr, zfeat)
    z = _tc3(aggH, W1, W2, b1.reshape(1, 32), b2.reshape(1, 32), degp, noise)
    adj_rec = _decoder(z)
    return (adj_rec, z)
